# Initial kernel scaffold; baseline (speedup 1.0000x reference)
import jax
import jax.numpy as jnp
from jax.experimental import pallas as pl


def kernel(inputs, tables):
    raise NotImplementedError("write your pallas kernel here")


# speedup vs baseline: 1.0459x; 1.0459x over previous
import jax
import jax.numpy as jnp
from jax import lax
from jax.experimental import pallas as pl
from jax.experimental.pallas import tpu as pltpu
from jax.experimental.pallas import tpu_sc as plsc

N_FIELDS = 26
VOCAB = 100000
DIM = 32
BATCH = 4096
NC, NS = 2, 16
NW = NC * NS
ROWS = BATCH // NW


def _body(tables, idx_t, *refs):
    outs = refs[:N_FIELDS]
    idx_v, rows_v, sem = refs[N_FIELDS:]
    wid = lax.axis_index("s") * NC + lax.axis_index("c")
    base = wid * ROWS
    pltpu.sync_copy(idx_t.at[:, pl.ds(base, ROWS)], idx_v)
    for j in range(N_FIELDS):
        pltpu.async_copy(tables.at[j].at[idx_v.at[j]], rows_v, sem).wait()
        pltpu.sync_copy(rows_v, outs[j].at[pl.ds(base, ROWS)])


def kernel(inputs, tables):
    idx_t = inputs.T
    mesh = plsc.VectorSubcoreMesh(
        core_axis_name="c", subcore_axis_name="s",
        num_cores=NC, num_subcores=NS)
    k = pl.kernel(
        _body,
        out_type=[jax.ShapeDtypeStruct((BATCH, DIM), jnp.float32)] * N_FIELDS,
        mesh=mesh,
        scratch_types=[
            pltpu.VMEM((N_FIELDS, ROWS), jnp.int32),
            pltpu.VMEM((ROWS, DIM), jnp.float32),
            pltpu.SemaphoreType.DMA,
        ],
        compiler_params=pltpu.CompilerParams(use_tc_tiling_on_sc=False),
    )
    outs = k(tables, idx_t)
    return tuple(o.reshape(-1) for o in outs)


# speedup vs baseline: 1.0596x; 1.0131x over previous
import jax
import jax.numpy as jnp
from jax import lax
from jax.experimental import pallas as pl
from jax.experimental.pallas import tpu as pltpu
from jax.experimental.pallas import tpu_sc as plsc

N_FIELDS = 26
VOCAB = 100000
DIM = 32
BATCH = 4096
NC, NS = 2, 16
NW = NC * NS
ROWS = BATCH // NW
K = 4


def _body(tables, idx_t, *refs):
    outs = refs[:N_FIELDS]
    idx_v = refs[N_FIELDS]
    bufs = refs[N_FIELDS + 1:N_FIELDS + 1 + K]
    gsems = refs[N_FIELDS + 1 + K:N_FIELDS + 1 + 2 * K]
    wsems = refs[N_FIELDS + 1 + 2 * K:N_FIELDS + 1 + 3 * K]
    wid = lax.axis_index("s") * NC + lax.axis_index("c")
    base = wid * ROWS
    pltpu.sync_copy(idx_t.at[:, pl.ds(base, ROWS)], idx_v)

    def gather(j):
        b = j % K
        return pltpu.async_copy(tables.at[j].at[idx_v.at[j]], bufs[b], gsems[b])

    gd = [None] * N_FIELDS
    wd = [None] * N_FIELDS
    for j in range(K):
        gd[j] = gather(j)
    for j in range(N_FIELDS):
        b = j % K
        gd[j].wait()
        wd[j] = pltpu.async_copy(bufs[b], outs[j].at[pl.ds(base, ROWS)], wsems[b])
        jn = j + K
        if jn < N_FIELDS:
            wd[j].wait()
            gd[jn] = gather(jn)
    for j in range(N_FIELDS - K, N_FIELDS):
        wd[j].wait()


def kernel(inputs, tables):
    idx_t = inputs.T
    mesh = plsc.VectorSubcoreMesh(
        core_axis_name="c", subcore_axis_name="s",
        num_cores=NC, num_subcores=NS)
    k = pl.kernel(
        _body,
        out_type=[jax.ShapeDtypeStruct((BATCH, DIM), jnp.float32)] * N_FIELDS,
        mesh=mesh,
        scratch_types=(
            [pltpu.VMEM((N_FIELDS, ROWS), jnp.int32)]
            + [pltpu.VMEM((ROWS, DIM), jnp.float32)] * K
            + [pltpu.SemaphoreType.DMA] * (2 * K)
        ),
        compiler_params=pltpu.CompilerParams(use_tc_tiling_on_sc=False),
    )
    outs = k(tables, idx_t)
    return tuple(o.reshape(-1) for o in outs)


# speedup vs baseline: 1.0615x; 1.0018x over previous
import jax
import jax.numpy as jnp
from jax import lax
from jax.experimental import pallas as pl
from jax.experimental.pallas import tpu as pltpu
from jax.experimental.pallas import tpu_sc as plsc

N_FIELDS = 26
VOCAB = 100000
DIM = 32
BATCH = 4096
NC, NS = 2, 16
NW = NC * NS
ROWS = BATCH // NW
K = 4
L = 16


def _body(tables, idx_t, *refs):
    outs = refs[:N_FIELDS]
    idx_v = refs[N_FIELDS]
    bufs = refs[N_FIELDS + 1:N_FIELDS + 1 + K]
    gsems = refs[N_FIELDS + 1 + K:N_FIELDS + 1 + 2 * K]
    wsems = refs[N_FIELDS + 1 + 2 * K:N_FIELDS + 1 + 3 * K]
    wid = lax.axis_index("s") * NC + lax.axis_index("c")
    base = wid * ROWS
    pltpu.sync_copy(idx_t.at[:, pl.ds(base, ROWS)], idx_v)

    def gather(j):
        b = j % K
        for k in range(ROWS // L):
            iv = idx_v[j, pl.ds(k * L, L)]
            pltpu.async_copy(tables.at[j].at[iv], bufs[b].at[pl.ds(k * L, L)],
                             gsems[b])

    def drain_gather(j):
        b = j % K
        pltpu.make_async_copy(outs[j].at[pl.ds(base, ROWS)], bufs[b],
                              gsems[b]).wait()

    wd = [None] * N_FIELDS
    for j in range(K):
        gather(j)
    for j in range(N_FIELDS):
        b = j % K
        drain_gather(j)
        wd[j] = pltpu.async_copy(bufs[b], outs[j].at[pl.ds(base, ROWS)],
                                 wsems[b])
        jn = j + K
        if jn < N_FIELDS:
            wd[j].wait()
            gather(jn)
    for j in range(N_FIELDS - K, N_FIELDS):
        wd[j].wait()


def kernel(inputs, tables):
    idx_t = inputs.T
    mesh = plsc.VectorSubcoreMesh(
        core_axis_name="c", subcore_axis_name="s",
        num_cores=NC, num_subcores=NS)
    k = pl.kernel(
        _body,
        out_type=[jax.ShapeDtypeStruct((BATCH, DIM), jnp.float32)] * N_FIELDS,
        mesh=mesh,
        scratch_types=(
            [pltpu.VMEM((N_FIELDS, ROWS), jnp.int32)]
            + [pltpu.VMEM((ROWS, DIM), jnp.float32)] * K
            + [pltpu.SemaphoreType.DMA] * (2 * K)
        ),
        compiler_params=pltpu.CompilerParams(use_tc_tiling_on_sc=False),
    )
    outs = k(tables, idx_t)
    return tuple(o.reshape(-1) for o in outs)


# speedup vs baseline: 1.9801x; 1.8654x over previous
import jax
import jax.numpy as jnp
from jax import lax
from jax.experimental import pallas as pl
from jax.experimental.pallas import tpu as pltpu
from jax.experimental.pallas import tpu_sc as plsc

N_FIELDS = 26
VOCAB = 100000
DIM = 32
BATCH = 4096
NC, NS = 2, 16
NW = NC * NS
ROWS = BATCH // NW
L = 16


def _body(tf, idx_t, *refs):
    outs = refs[:N_FIELDS]
    idx_v = refs[N_FIELDS]
    tbufs = refs[N_FIELDS + 1:N_FIELDS + 3]
    gsems = refs[N_FIELDS + 3:N_FIELDS + 5]
    wsems = refs[N_FIELDS + 5:N_FIELDS + 7]
    wid = lax.axis_index("s") * NC + lax.axis_index("c")
    base = wid * ROWS
    pltpu.sync_copy(idx_t.at[:, pl.ds(base, ROWS)], idx_v)
    coff_lo = lax.broadcasted_iota(jnp.int32, (L,), 0) * VOCAB
    coff_hi = coff_lo + L * VOCAB

    def gather(j):
        t = tbufs[j % 2]

        def chunk(k, _):
            v = idx_v[j, pl.ds(k * L, L)]
            for l in range(L):
                r = v[l]
                pltpu.async_copy(tf.at[j].at[coff_lo + r],
                                 t.at[k * L + l, pl.ds(0, L)], gsems[j % 2])
                pltpu.async_copy(tf.at[j].at[coff_hi + r],
                                 t.at[k * L + l, pl.ds(L, L)], gsems[j % 2])
            return _

        lax.fori_loop(0, ROWS // L, chunk, 0)

    def drain_gather(j):
        pltpu.make_async_copy(outs[j].at[pl.ds(base, ROWS)],
                              tbufs[j % 2], gsems[j % 2]).wait()

    wd = [None] * N_FIELDS
    gather(0)
    for j in range(N_FIELDS):
        drain_gather(j)
        if j + 1 < N_FIELDS:
            gather(j + 1)
        if j >= 2:
            wd[j - 2].wait()
        wd[j] = pltpu.async_copy(tbufs[j % 2],
                                 outs[j].at[pl.ds(base, ROWS)], wsems[j % 2])
    wd[N_FIELDS - 2].wait()
    wd[N_FIELDS - 1].wait()


def kernel(inputs, tables):
    idx_t = inputs.T
    tf = jnp.transpose(tables, (0, 2, 1)).reshape(N_FIELDS, DIM * VOCAB)
    mesh = plsc.VectorSubcoreMesh(
        core_axis_name="c", subcore_axis_name="s",
        num_cores=NC, num_subcores=NS)
    k = pl.kernel(
        _body,
        out_type=[jax.ShapeDtypeStruct((BATCH, DIM), jnp.float32)] * N_FIELDS,
        mesh=mesh,
        scratch_types=(
            [pltpu.VMEM((N_FIELDS, ROWS), jnp.int32)]
            + [pltpu.VMEM((ROWS, DIM), jnp.float32)] * 2
            + [pltpu.SemaphoreType.DMA] * 4
        ),
        compiler_params=pltpu.CompilerParams(use_tc_tiling_on_sc=False,
                                             needs_layout_passes=False),
    )
    outs = k(tf, idx_t)
    return tuple(o.reshape(-1) for o in outs)
